# baseline (speedup 1.0000x reference)
import jax
import jax.numpy as jnp
from jax.experimental import pallas as pl
from jax.experimental.pallas import tpu as pltpu

_N = 2048
_IN = 16384
_K = 40
_BLK = 128
_NB = _N // _BLK


def _fused_body(inp_ref, conn_ref, out_ref, ov_col, ov_row, rank_col):
    s = pl.program_id(0)
    base = s * _BLK

    @pl.when(s == 0)
    def _init():
        ov_col[:] = jnp.full((_N, 1), -1.0, jnp.float32)
        ov_row[:] = jnp.full((1, _N), -1.0, jnp.float32)

    ov = jnp.sum(conn_ref[:] * inp_ref[:], axis=1)
    ov_r = ov.reshape(1, _BLK)
    ov_c = ov.reshape(_BLK, 1)
    ov_row[:, pl.ds(base, _BLK)] = ov_r
    ov_col[pl.ds(base, _BLK), :] = ov_c

    orow = ov_row[:]
    jcol = jax.lax.broadcasted_iota(jnp.int32, (_BLK, _N), 1)
    irow = jax.lax.broadcasted_iota(jnp.int32, (_BLK, _N), 0) + base
    gt = (orow > ov_c)
    eqb = (orow == ov_c) & (jcol < irow)
    rank_new = jnp.sum(
        jnp.where(gt | eqb, 1.0, 0.0), axis=1, keepdims=True
    )
    rank_col[pl.ds(base, _BLK), :] = rank_new

    oc_all = ov_col[:]
    irow2 = jax.lax.broadcasted_iota(jnp.int32, (_N, _BLK), 0)
    upd = jnp.where((ov_r > oc_all) & (irow2 < base), 1.0, 0.0)
    rank_col[:] = rank_col[:] + jnp.sum(upd, axis=1, keepdims=True)

    @pl.when(s == _NB - 1)
    def _mask():
        out_ref[:] = (rank_col[:] < float(_K)).astype(jnp.float32)


def kernel(input_vector, connections):
    mask = pl.pallas_call(
        _fused_body,
        grid=(_NB,),
        in_specs=[
            pl.BlockSpec((1, _IN), lambda i: (0, 0)),
            pl.BlockSpec((_BLK, _IN), lambda i: (i, 0)),
        ],
        out_specs=pl.BlockSpec((_N, 1), lambda i: (0, 0)),
        out_shape=jax.ShapeDtypeStruct((_N, 1), jnp.float32),
        scratch_shapes=[
            pltpu.VMEM((_N, 1), jnp.float32),
            pltpu.VMEM((1, _N), jnp.float32),
            pltpu.VMEM((_N, 1), jnp.float32),
        ],
    )(input_vector.reshape(1, _IN), connections)
    return mask.reshape(_N)


# speedup vs baseline: 5.5108x; 5.5108x over previous
import jax
import jax.numpy as jnp
from jax.experimental import pallas as pl
from jax.experimental.pallas import tpu as pltpu

_N = 2048
_IN = 16384
_K = 40
_BLK = 128
_NB = _N // _BLK


def _fused_body(inp_ref, conn_ref, out_ref, ov_row, rank_row):
    s = pl.program_id(0)
    base = s * _BLK

    @pl.when(s == 0)
    def _init():
        ov_row[:] = jnp.full((1, _N), -1.0, jnp.float32)

    ov = jnp.sum(conn_ref[:] * inp_ref[:], axis=1)
    ov_r = ov.reshape(1, _BLK)
    ov_row[:, pl.ds(base, _BLK)] = ov_r
    ov_c = ov.reshape(_BLK, 1)
    orow = ov_row[:]

    gt = (orow > ov_c)
    jcol = jax.lax.broadcasted_iota(jnp.int32, (1, _N), 1)
    eq_lo = (orow == ov_c) & (jcol < base)
    tri = (
        jax.lax.broadcasted_iota(jnp.int32, (_BLK, _BLK), 1)
        < jax.lax.broadcasted_iota(jnp.int32, (_BLK, _BLK), 0)
    )
    eq_dg = (ov_r == ov_c) & tri
    rank_new = jnp.sum(
        jnp.where(gt | eq_lo, 1.0, 0.0), axis=1
    ) + jnp.sum(jnp.where(eq_dg, 1.0, 0.0), axis=1)
    upd = jnp.sum(
        jnp.where((ov_c > orow) & (jcol < base), 1.0, 0.0), axis=0
    )
    new_rank = rank_row[:] + upd.reshape(1, _N)
    rank_row[:] = new_rank
    rank_row[:, pl.ds(base, _BLK)] = rank_new.reshape(1, _BLK)

    @pl.when(s == _NB - 1)
    def _mask():
        out_ref[:] = (rank_row[:] < float(_K)).astype(jnp.float32)


def kernel(input_vector, connections):
    mask = pl.pallas_call(
        _fused_body,
        grid=(_NB,),
        in_specs=[
            pl.BlockSpec((1, _IN), lambda i: (0, 0)),
            pl.BlockSpec((_BLK, _IN), lambda i: (i, 0)),
        ],
        out_specs=pl.BlockSpec((1, _N), lambda i: (0, 0)),
        out_shape=jax.ShapeDtypeStruct((1, _N), jnp.float32),
        scratch_shapes=[
            pltpu.VMEM((1, _N), jnp.float32),
            pltpu.VMEM((1, _N), jnp.float32),
        ],
    )(input_vector.reshape(1, _IN), connections)
    return mask.reshape(_N)


# speedup vs baseline: 10.7220x; 1.9456x over previous
import jax
import jax.numpy as jnp
from jax.experimental import pallas as pl
from jax.experimental.pallas import tpu as pltpu

_N = 2048
_IN = 16384
_K = 40
_BLK = 128
_NB = _N // _BLK


def _fused_body(inp_ref, conn_ref, out_ref, ov_blk, ov_col):
    s = pl.program_id(0)
    ov = jnp.sum(conn_ref[:] * inp_ref[:], axis=1)
    ov_blk[pl.ds(s, 1), :] = ov.reshape(1, _BLK)
    ov_col[pl.ds(s * _BLK, _BLK), :] = ov.reshape(_BLK, 1)

    @pl.when(s == _NB - 1)
    def _rank_and_mask():
        orow = ov_blk[:].reshape(1, _N)
        tri = (
            jax.lax.broadcasted_iota(jnp.int32, (_BLK, _BLK), 1)
            < jax.lax.broadcasted_iota(jnp.int32, (_BLK, _BLK), 0)
        )
        for b in range(_NB):
            lo, hi = b * _BLK, (b + 1) * _BLK
            oc = ov_col[lo:hi, :]
            rank = jnp.zeros((_BLK, 1), jnp.float32)
            if b > 0:
                rank = jnp.sum(
                    jnp.where(orow[:, :lo] >= oc, 1.0, 0.0),
                    axis=1, keepdims=True,
                )
            rank = rank + jnp.sum(
                jnp.where(orow[:, lo:] > oc, 1.0, 0.0),
                axis=1, keepdims=True,
            )
            rank = rank + jnp.sum(
                jnp.where((orow[:, lo:hi] == oc) & tri, 1.0, 0.0),
                axis=1, keepdims=True,
            )
            out_ref[lo:hi, :] = (rank < float(_K)).astype(jnp.float32)


def kernel(input_vector, connections):
    mask = pl.pallas_call(
        _fused_body,
        grid=(_NB,),
        in_specs=[
            pl.BlockSpec((1, _IN), lambda i: (0, 0)),
            pl.BlockSpec((_BLK, _IN), lambda i: (i, 0)),
        ],
        out_specs=pl.BlockSpec((_N, 1), lambda i: (0, 0)),
        out_shape=jax.ShapeDtypeStruct((_N, 1), jnp.float32),
        scratch_shapes=[
            pltpu.VMEM((_NB, _BLK), jnp.float32),
            pltpu.VMEM((_N, 1), jnp.float32),
        ],
    )(input_vector.reshape(1, _IN), connections)
    return mask.reshape(_N)
